# Initial kernel scaffold; baseline (speedup 1.0000x reference)
import functools

import jax
import jax.numpy as jnp
from jax import lax
from jax.experimental import pallas as pl
from jax.experimental.pallas import tpu as pltpu
from jax.experimental.pallas import tpu_sc as plsc

_NUM_CORES = 2
_NUM_SUBCORES = 16
_NW = _NUM_CORES * _NUM_SUBCORES
_LANES = 16

_C = 128
_ROWS_PER_W = 320
_NPAD = _ROWS_PER_W * _NW
_CHUNK = 2000
_GB = 64
_WLCAP = _GB + 16
_NEG = -3.0e38

_BLK = 1024


def _leaky(v):
  return jnp.where(v >= 0, v, 0.01 * v)


def _ab_body(x_ref, p8_ref, wfc_ref, wf38_ref, bf_ref, wh1_ref, bh1_ref,
             wh28_ref, bh28_ref, a_ref, b_ref):
  x = x_ref[...]
  p8 = p8_ref[...]
  h = _leaky(jnp.dot(x, wh1_ref[...], preferred_element_type=jnp.float32)
             + bh1_ref[...])
  d8 = jnp.tanh(jnp.dot(h, wh28_ref[...], preferred_element_type=jnp.float32)
                + bh28_ref[...])
  wf38 = wf38_ref[...]
  a_ref[...] = (jnp.dot(x, wfc_ref[...], preferred_element_type=jnp.float32)
                + jnp.dot(p8, wf38, preferred_element_type=jnp.float32)
                + bf_ref[...])
  b_ref[...] = jnp.dot(d8 - p8, wf38, preferred_element_type=jnp.float32)


def _ab_call(x, p8, wfc, wf38, bf2, wh1, bh12, wh28, bh28):
  n = x.shape[0]
  grid = (n // _BLK,)
  row = lambda i: (i, 0)
  fix = lambda i: (0, 0)
  return pl.pallas_call(
      _ab_body,
      grid=grid,
      in_specs=[
          pl.BlockSpec((_BLK, _C), row),
          pl.BlockSpec((_BLK, 8), row),
          pl.BlockSpec((_C, _C), fix),
          pl.BlockSpec((8, _C), fix),
          pl.BlockSpec((1, _C), fix),
          pl.BlockSpec((_C, _C), fix),
          pl.BlockSpec((1, _C), fix),
          pl.BlockSpec((_C, 8), fix),
          pl.BlockSpec((1, 8), fix),
      ],
      out_specs=[pl.BlockSpec((_BLK, _C), row), pl.BlockSpec((_BLK, _C), row)],
      out_shape=[jax.ShapeDtypeStruct((n, _C), jnp.float32)] * 2,
  )(x, p8, wfc, wf38, bf2, wh1, bh12, wh28, bh28)


def _segmax_kernel(a_hbm, src_hbm, dst_hbm, out_hbm,
                   dstv, srcv, wls, wld, rows, acc, sem):
  wid = lax.axis_index("s") * _NUM_CORES + lax.axis_index("c")
  lo = wid * _ROWS_PER_W
  dummy = _ROWS_PER_W

  neg = jnp.full((_LANES,), _NEG, jnp.float32)

  @pl.loop(0, (_ROWS_PER_W + 8) * _C, step=_LANES)
  def _(i):
    acc[pl.ds(i, _LANES)] = neg

  zeros = jnp.zeros((_LANES,), jnp.int32)
  dums = jnp.full((_LANES,), dummy, jnp.int32)

  @pl.loop(0, _WLCAP, step=_LANES)
  def _(i):
    wls[pl.ds(i, _LANES)] = zeros
    wld[pl.ds(i, _LANES)] = dums

  nchunks = src_hbm.shape[0] // _CHUNK
  ngroups = _CHUNK // _LANES

  def drain(cnt):
    pltpu.async_copy(a_hbm.at[wls.at[pl.ds(0, _GB)]], rows, sem).wait()

    @pl.loop(0, _GB)
    def _(j):
      d = wld[j]
      base = d * _C
      for c in range(_C // _LANES):
        sl = pl.ds(base + c * _LANES, _LANES)
        acc[sl] = jnp.maximum(acc[sl], rows[j, pl.ds(c * _LANES, _LANES)])

    wls[pl.ds(0, _LANES)] = wls[pl.ds(_GB, _LANES)]
    wld[pl.ds(0, _LANES)] = wld[pl.ds(_GB, _LANES)]
    return cnt - _GB

  def group_body(gi, cnt, dst_chunk, src_chunk):
    d16 = dst_chunk[pl.ds(gi * _LANES, _LANES)]
    s16 = src_chunk[pl.ds(gi * _LANES, _LANES)]
    ld = d16 - lo
    mask = (ld >= 0) & (ld < _ROWS_PER_W)
    mi = jnp.where(mask, 1, 0).astype(jnp.int32)
    pos = plsc.cumsum(mi) + (cnt - 1)
    plsc.store_scatter(wls, [pos], s16, mask)
    plsc.store_scatter(wld, [pos], ld, mask)
    cnt = cnt + jnp.sum(mi)
    return lax.cond(cnt >= _GB, drain, lambda c: c, cnt)

  def chunk_body(ci, cnt):
    pltpu.sync_copy(dst_hbm.at[pl.ds(ci * _CHUNK, _CHUNK)], dstv)
    pltpu.sync_copy(src_hbm.at[pl.ds(ci * _CHUNK, _CHUNK)], srcv)
    return lax.fori_loop(
        0, ngroups, lambda g, c: group_body(g, c, dstv, srcv), cnt)

  cnt = lax.fori_loop(0, nchunks, chunk_body, jnp.int32(0))
  _ = drain(cnt)

  pltpu.sync_copy(acc.at[pl.ds(0, _ROWS_PER_W * _C)],
                  out_hbm.at[pl.ds(lo * _C, _ROWS_PER_W * _C)])


def _segmax_call(a, src, dst):
  mesh = plsc.VectorSubcoreMesh(core_axis_name="c", subcore_axis_name="s")
  kern = pl.kernel(
      _segmax_kernel,
      mesh=mesh,
      out_type=jax.ShapeDtypeStruct((_NPAD * _C,), jnp.float32),
      scratch_types=[
          pltpu.VMEM((_CHUNK,), jnp.int32),
          pltpu.VMEM((_CHUNK,), jnp.int32),
          pltpu.VMEM((_WLCAP,), jnp.int32),
          pltpu.VMEM((_WLCAP,), jnp.int32),
          pltpu.VMEM((_GB, _C), jnp.float32),
          pltpu.VMEM(((_ROWS_PER_W + 8) * _C,), jnp.float32),
          pltpu.SemaphoreType.DMA,
      ],
  )
  return kern(a, src, dst)


def _out_body(s_ref, b_ref, x_ref, wg1_ref, bg1_ref, wg2_ref, bg2_ref, o_ref):
  s = s_ref[...]
  agg = _leaky(s + b_ref[...])
  agg = jnp.where(s < -1.0e38, 0.0, agg)
  g = _leaky(jnp.dot(agg, wg1_ref[...], preferred_element_type=jnp.float32)
             + bg1_ref[...])
  o_ref[...] = (x_ref[...]
                + jnp.dot(g, wg2_ref[...], preferred_element_type=jnp.float32)
                + bg2_ref[...])


def _out_call(s, b, x, wg1, bg12, wg2, bg22):
  n = x.shape[0]
  row = lambda i: (i, 0)
  fix = lambda i: (0, 0)
  return pl.pallas_call(
      _out_body,
      grid=(n // _BLK,),
      in_specs=[
          pl.BlockSpec((_BLK, _C), row),
          pl.BlockSpec((_BLK, _C), row),
          pl.BlockSpec((_BLK, _C), row),
          pl.BlockSpec((_C, _C), fix),
          pl.BlockSpec((1, _C), fix),
          pl.BlockSpec((_C, _C), fix),
          pl.BlockSpec((1, _C), fix),
      ],
      out_specs=pl.BlockSpec((_BLK, _C), row),
      out_shape=jax.ShapeDtypeStruct((n, _C), jnp.float32),
  )(s, b, x, wg1, bg12, wg2, bg22)


def kernel(x, pos, edge_index, Wh1, bh1, Wh2, bh2, Wf, bf, Wg1, bg1, Wg2, bg2):
  n = x.shape[0]
  e = edge_index.shape[1]

  xp = jnp.pad(x, ((0, _NPAD - n), (0, 0)))
  p8 = jnp.pad(pos, ((0, _NPAD - n), (0, 5)))
  wf38 = jnp.pad(Wf[:3], ((0, 5), (0, 0)))
  wfc = Wf[3:]
  wh28 = jnp.pad(Wh2, ((0, 0), (0, 5)))
  bh28 = jnp.pad(bh2, (0, 5)).reshape(1, 8)

  a, b = _ab_call(xp, p8, wfc, wf38, bf.reshape(1, _C), Wh1,
                  bh1.reshape(1, _C), wh28, bh28)

  src = edge_index[0].astype(jnp.int32)
  dst = edge_index[1].astype(jnp.int32)
  epad = (-e) % _CHUNK
  if epad:
    src = jnp.pad(src, (0, epad))
    dst = jnp.pad(dst, (0, epad), constant_values=_NPAD - 1)

  s_flat = _segmax_call(a, src, dst)
  s = s_flat.reshape(_NPAD, _C)

  out = _out_call(s, b, xp, Wg1, bg1.reshape(1, _C), Wg2, bg2.reshape(1, _C))
  return out[:n]


# speedup vs baseline: 3.1613x; 3.1613x over previous
import dataclasses
import functools

import jax
import jax.numpy as jnp
from jax import lax
from jax.experimental import pallas as pl
from jax.experimental.pallas import tpu as pltpu
from jax.experimental.pallas import tpu_sc as plsc

_NUM_CORES = 2
_NUM_SUBCORES = 16
_NW = _NUM_CORES * _NUM_SUBCORES
_LANES = 16

_C = 128
_ROWS_PER_W = 320
_NPAD = _ROWS_PER_W * _NW
_CHUNK = 2000
_GB = 64
_WLCAP = _GB + 16
_NEG = -3.0e38

_BLK = 1024


def _leaky(v):
  return jnp.where(v >= 0, v, 0.01 * v)


def _ab_body(x_ref, p8_ref, wfc_ref, wf38_ref, bf_ref, wh1_ref, bh1_ref,
             wh28_ref, bh28_ref, a_ref, b_ref):
  x = x_ref[...]
  p8 = p8_ref[...]
  h = _leaky(jnp.dot(x, wh1_ref[...], preferred_element_type=jnp.float32)
             + bh1_ref[...])
  d8 = jnp.tanh(jnp.dot(h, wh28_ref[...], preferred_element_type=jnp.float32)
                + bh28_ref[...])
  wf38 = wf38_ref[...]
  a_ref[...] = (jnp.dot(x, wfc_ref[...], preferred_element_type=jnp.float32)
                + jnp.dot(p8, wf38, preferred_element_type=jnp.float32)
                + bf_ref[...])
  b_ref[...] = jnp.dot(d8 - p8, wf38, preferred_element_type=jnp.float32)


def _ab_call(x, p8, wfc, wf38, bf2, wh1, bh12, wh28, bh28):
  n = x.shape[0]
  grid = (n // _BLK,)
  row = lambda i: (i, 0)
  fix = lambda i: (0, 0)
  return pl.pallas_call(
      _ab_body,
      grid=grid,
      in_specs=[
          pl.BlockSpec((_BLK, _C), row),
          pl.BlockSpec((_BLK, 8), row),
          pl.BlockSpec((_C, _C), fix),
          pl.BlockSpec((8, _C), fix),
          pl.BlockSpec((1, _C), fix),
          pl.BlockSpec((_C, _C), fix),
          pl.BlockSpec((1, _C), fix),
          pl.BlockSpec((_C, 8), fix),
          pl.BlockSpec((1, 8), fix),
      ],
      out_specs=[pl.BlockSpec((_BLK, _C), row), pl.BlockSpec((_BLK, _C), row)],
      out_shape=[jax.ShapeDtypeStruct((n, _C), jnp.float32)] * 2,
  )(x, p8, wfc, wf38, bf2, wh1, bh12, wh28, bh28)


def _segmax_kernel(a_hbm, src_hbm, dst_hbm, out_hbm,
                   dstv, srcv, wls, wld, rows, acc, sem):
  wid = lax.axis_index("s") * _NUM_CORES + lax.axis_index("c")
  lo = wid * _ROWS_PER_W
  dummy = _ROWS_PER_W

  neg = jnp.full((_LANES,), _NEG, jnp.float32)

  @pl.loop(0, (_ROWS_PER_W + 8) * _C, step=_LANES)
  def _(i):
    acc[pl.ds(i, _LANES)] = neg

  zeros = jnp.zeros((_LANES,), jnp.int32)
  dums = jnp.full((_LANES,), dummy, jnp.int32)

  @pl.loop(0, _WLCAP, step=_LANES)
  def _(i):
    wls[pl.ds(i, _LANES)] = zeros
    wld[pl.ds(i, _LANES)] = dums

  nchunks = src_hbm.shape[0] // _CHUNK
  ngroups = _CHUNK // _LANES

  def drain(cnt):
    pltpu.async_copy(a_hbm.at[wls.at[pl.ds(0, _GB)]], rows, sem).wait()

    @pl.loop(0, _GB // _LANES)
    def _(g):
      dvec = wld[pl.ds(g * _LANES, _LANES)]
      for l in range(_LANES):
        base = dvec[l] * _C
        j = g * _LANES + l
        for c in range(_C // _LANES):
          sl = pl.ds(base + c * _LANES, _LANES)
          acc[sl] = jnp.maximum(acc[sl], rows[j, pl.ds(c * _LANES, _LANES)])

    wls[pl.ds(0, _LANES)] = wls[pl.ds(_GB, _LANES)]
    wld[pl.ds(0, _LANES)] = wld[pl.ds(_GB, _LANES)]
    return cnt - _GB

  def group_body(gi, cnt, dst_chunk, src_chunk):
    d16 = dst_chunk[pl.ds(gi * _LANES, _LANES)]
    s16 = src_chunk[pl.ds(gi * _LANES, _LANES)]
    ld = d16 - lo
    mask = (ld >= 0) & (ld < _ROWS_PER_W)
    mi = jnp.where(mask, 1, 0).astype(jnp.int32)
    pos = plsc.cumsum(mi) + (cnt - 1)
    plsc.store_scatter(wls, [pos], s16, mask=mask)
    plsc.store_scatter(wld, [pos], ld, mask=mask)
    cnt = cnt + jnp.sum(mi)
    return lax.cond(cnt >= _GB, drain, lambda c: c, cnt)

  def chunk_body(ci, cnt):
    pltpu.sync_copy(dst_hbm.at[pl.ds(ci * _CHUNK, _CHUNK)], dstv)
    pltpu.sync_copy(src_hbm.at[pl.ds(ci * _CHUNK, _CHUNK)], srcv)
    return lax.fori_loop(
        0, ngroups, lambda g, c: group_body(g, c, dstv, srcv), cnt)

  cnt = lax.fori_loop(0, nchunks, chunk_body, jnp.int32(0))
  _ = drain(cnt)

  pltpu.sync_copy(acc.at[pl.ds(0, _ROWS_PER_W * _C)],
                  out_hbm.at[pl.ds(lo * _C, _ROWS_PER_W * _C)])


def _segmax_call(a, src, dst):
  mesh = plsc.VectorSubcoreMesh(core_axis_name="c", subcore_axis_name="s")
  cp = pltpu.CompilerParams()
  if "needs_layout_passes" in pltpu.CompilerParams.__dataclass_fields__:
    cp = dataclasses.replace(cp, needs_layout_passes=False)
  kern = pl.kernel(
      _segmax_kernel,
      mesh=mesh,
      compiler_params=cp,
      out_type=jax.ShapeDtypeStruct((_NPAD * _C,), jnp.float32),
      scratch_types=[
          pltpu.VMEM((_CHUNK,), jnp.int32),
          pltpu.VMEM((_CHUNK,), jnp.int32),
          pltpu.VMEM((_WLCAP,), jnp.int32),
          pltpu.VMEM((_WLCAP,), jnp.int32),
          pltpu.VMEM((_GB, _C), jnp.float32),
          pltpu.VMEM(((_ROWS_PER_W + 8) * _C,), jnp.float32),
          pltpu.SemaphoreType.DMA,
      ],
  )
  return kern(a, src, dst)


def _out_body(s_ref, b_ref, x_ref, wg1_ref, bg1_ref, wg2_ref, bg2_ref, o_ref):
  s = s_ref[...]
  agg = _leaky(s + b_ref[...])
  agg = jnp.where(s < -1.0e38, 0.0, agg)
  g = _leaky(jnp.dot(agg, wg1_ref[...], preferred_element_type=jnp.float32)
             + bg1_ref[...])
  o_ref[...] = (x_ref[...]
                + jnp.dot(g, wg2_ref[...], preferred_element_type=jnp.float32)
                + bg2_ref[...])


def _out_call(s, b, x, wg1, bg12, wg2, bg22):
  n = x.shape[0]
  row = lambda i: (i, 0)
  fix = lambda i: (0, 0)
  return pl.pallas_call(
      _out_body,
      grid=(n // _BLK,),
      in_specs=[
          pl.BlockSpec((_BLK, _C), row),
          pl.BlockSpec((_BLK, _C), row),
          pl.BlockSpec((_BLK, _C), row),
          pl.BlockSpec((_C, _C), fix),
          pl.BlockSpec((1, _C), fix),
          pl.BlockSpec((_C, _C), fix),
          pl.BlockSpec((1, _C), fix),
      ],
      out_specs=pl.BlockSpec((_BLK, _C), row),
      out_shape=jax.ShapeDtypeStruct((n, _C), jnp.float32),
  )(s, b, x, wg1, bg12, wg2, bg22)


def kernel(x, pos, edge_index, Wh1, bh1, Wh2, bh2, Wf, bf, Wg1, bg1, Wg2, bg2):
  n = x.shape[0]
  e = edge_index.shape[1]

  xp = jnp.pad(x, ((0, _NPAD - n), (0, 0)))
  p8 = jnp.pad(pos, ((0, _NPAD - n), (0, 5)))
  wf38 = jnp.pad(Wf[:3], ((0, 5), (0, 0)))
  wfc = Wf[3:]
  wh28 = jnp.pad(Wh2, ((0, 0), (0, 5)))
  bh28 = jnp.pad(bh2, (0, 5)).reshape(1, 8)

  a, b = _ab_call(xp, p8, wfc, wf38, bf.reshape(1, _C), Wh1,
                  bh1.reshape(1, _C), wh28, bh28)

  src = edge_index[0].astype(jnp.int32)
  dst = edge_index[1].astype(jnp.int32)
  epad = (-e) % _CHUNK
  if epad:
    src = jnp.pad(src, (0, epad))
    dst = jnp.pad(dst, (0, epad), constant_values=_NPAD - 1)

  s_flat = _segmax_call(a, src, dst)
  s = s_flat.reshape(_NPAD, _C)

  out = _out_call(s, b, xp, Wg1, bg1.reshape(1, _C), Wg2, bg2.reshape(1, _C))
  return out[:n]


# speedup vs baseline: 4.0592x; 1.2840x over previous
import dataclasses
import functools

import jax
import jax.numpy as jnp
from jax import lax
from jax.experimental import pallas as pl
from jax.experimental.pallas import tpu as pltpu
from jax.experimental.pallas import tpu_sc as plsc

_NUM_CORES = 2
_NUM_SUBCORES = 16
_NW = _NUM_CORES * _NUM_SUBCORES
_LANES = 16

_C = 128
_ROWS_PER_W = 320
_NPAD = _ROWS_PER_W * _NW
_CHUNK = 4000
_GB = 128
_WLCAP = _CHUNK + _GB + 16
_NEG = -3.0e38

_BLK = 1024


def _leaky(v):
  return jnp.where(v >= 0, v, 0.01 * v)


def _ab_body(x_ref, p8_ref, wfc_ref, wf38_ref, bf_ref, wh1_ref, bh1_ref,
             wh28_ref, bh28_ref, a_ref, b_ref):
  x = x_ref[...]
  p8 = p8_ref[...]
  h = _leaky(jnp.dot(x, wh1_ref[...], preferred_element_type=jnp.float32)
             + bh1_ref[...])
  d8 = jnp.tanh(jnp.dot(h, wh28_ref[...], preferred_element_type=jnp.float32)
                + bh28_ref[...])
  wf38 = wf38_ref[...]
  a_ref[...] = (jnp.dot(x, wfc_ref[...], preferred_element_type=jnp.float32)
                + jnp.dot(p8, wf38, preferred_element_type=jnp.float32)
                + bf_ref[...])
  b_ref[...] = jnp.dot(d8 - p8, wf38, preferred_element_type=jnp.float32)


def _ab_call(x, p8, wfc, wf38, bf2, wh1, bh12, wh28, bh28):
  n = x.shape[0]
  grid = (n // _BLK,)
  row = lambda i: (i, 0)
  fix = lambda i: (0, 0)
  return pl.pallas_call(
      _ab_body,
      grid=grid,
      in_specs=[
          pl.BlockSpec((_BLK, _C), row),
          pl.BlockSpec((_BLK, 8), row),
          pl.BlockSpec((_C, _C), fix),
          pl.BlockSpec((8, _C), fix),
          pl.BlockSpec((1, _C), fix),
          pl.BlockSpec((_C, _C), fix),
          pl.BlockSpec((1, _C), fix),
          pl.BlockSpec((_C, 8), fix),
          pl.BlockSpec((1, 8), fix),
      ],
      out_specs=[pl.BlockSpec((_BLK, _C), row), pl.BlockSpec((_BLK, _C), row)],
      out_shape=[jax.ShapeDtypeStruct((n, _C), jnp.float32)] * 2,
  )(x, p8, wfc, wf38, bf2, wh1, bh12, wh28, bh28)


def _segmax_kernel(a_hbm, src_hbm, dst_hbm, out_hbm,
                   dstv, srcv, wls, wld, rbuf0, rbuf1, acc,
                   semc, semr0, semr1):
  wid = lax.axis_index("s") * _NUM_CORES + lax.axis_index("c")
  lo = wid * _ROWS_PER_W
  dummy = _ROWS_PER_W

  neg = jnp.full((_LANES,), _NEG, jnp.float32)

  @pl.loop(0, (_ROWS_PER_W + 8) * _C, step=_LANES)
  def _(i):
    acc[pl.ds(i, _LANES)] = neg

  zeros = jnp.zeros((_LANES,), jnp.int32)
  dums = jnp.full((_LANES,), dummy, jnp.int32)

  @pl.loop(0, _WLCAP, step=_LANES)
  def _(i):
    wls[pl.ds(i, _LANES)] = zeros
    wld[pl.ds(i, _LANES)] = dums

  nchunks = src_hbm.shape[0] // _CHUNK
  ngroups = _CHUNK // _LANES

  def fire_chunk(ci):
    off = lax.rem(ci, 2) * _CHUNK
    pltpu.async_copy(dst_hbm.at[pl.ds(ci * _CHUNK, _CHUNK)],
                     dstv.at[pl.ds(off, _CHUNK)], semc)
    pltpu.async_copy(src_hbm.at[pl.ds(ci * _CHUNK, _CHUNK)],
                     srcv.at[pl.ds(off, _CHUNK)], semc)

  def wait_chunk():
    pltpu.make_async_copy(dst_hbm.at[pl.ds(0, _CHUNK)],
                          dstv.at[pl.ds(0, _CHUNK)], semc).wait()
    pltpu.make_async_copy(src_hbm.at[pl.ds(0, _CHUNK)],
                          srcv.at[pl.ds(0, _CHUNK)], semc).wait()

  def fire_batch(b, rbuf, semr):
    pltpu.async_copy(a_hbm.at[wls.at[pl.ds(b * _GB, _GB)]], rbuf, semr)

  def wait_batch(b, rbuf, semr):
    pltpu.make_async_copy(a_hbm.at[wls.at[pl.ds(b * _GB, _GB)]],
                          rbuf, semr).wait()

  def accum_batch(b, rbuf):
    @pl.loop(0, _GB // _LANES)
    def _(g):
      dvec = wld[pl.ds(b * _GB + g * _LANES, _LANES)]
      for l in range(_LANES):
        base = dvec[l] * _C
        j = g * _LANES + l
        for c in range(_C // _LANES):
          sl = pl.ds(base + c * _LANES, _LANES)
          acc[sl] = jnp.maximum(acc[sl], rbuf[j, pl.ds(c * _LANES, _LANES)])

  def scan_group(g, cnt, pbase):
    d16 = dstv[pl.ds(pbase + g * _LANES, _LANES)]
    ld = d16 - lo
    mask = (ld >= 0) & (ld < _ROWS_PER_W)
    mi = jnp.where(mask, 1, 0).astype(jnp.int32)
    csum = plsc.cumsum(mi)
    total = csum[_LANES - 1]

    @pl.when(total > 0)
    def _():
      s16 = srcv[pl.ds(pbase + g * _LANES, _LANES)]
      pos = csum + (cnt - 1)
      plsc.store_scatter(wls, [pos], s16, mask=mask)
      plsc.store_scatter(wld, [pos], ld, mask=mask)

    return cnt + total

  def drain_all(cnt):
    nb = cnt // _GB

    @pl.when(nb > 0)
    def _():
      fire_batch(0, rbuf0, semr0)

    def body(b, carry):
      def even(x):
        @pl.when(b + 1 < nb)
        def _():
          fire_batch(b + 1, rbuf1, semr1)
        wait_batch(b, rbuf0, semr0)
        accum_batch(b, rbuf0)
        return x

      def odd(x):
        @pl.when(b + 1 < nb)
        def _():
          fire_batch(b + 1, rbuf0, semr0)
        wait_batch(b, rbuf1, semr1)
        accum_batch(b, rbuf1)
        return x

      return lax.cond(lax.rem(b, 2) == 0, even, odd, carry)

    lax.fori_loop(0, nb, body, jnp.int32(0))

    for k in range(_GB // _LANES):
      wls[pl.ds(k * _LANES, _LANES)] = wls[pl.ds(nb * _GB + k * _LANES,
                                                 _LANES)]
      wld[pl.ds(k * _LANES, _LANES)] = wld[pl.ds(nb * _GB + k * _LANES,
                                                 _LANES)]
    return cnt - nb * _GB

  fire_chunk(0)

  def chunk_body(ci, cnt):
    wait_chunk()

    @pl.when(ci + 1 < nchunks)
    def _():
      fire_chunk(ci + 1)

    pbase = lax.rem(ci, 2) * _CHUNK
    cnt = lax.fori_loop(0, ngroups, lambda g, c: scan_group(g, c, pbase), cnt)
    return drain_all(cnt)

  _ = lax.fori_loop(0, nchunks, chunk_body, jnp.int32(0))

  fire_batch(0, rbuf0, semr0)
  wait_batch(0, rbuf0, semr0)
  accum_batch(0, rbuf0)

  pltpu.sync_copy(acc.at[pl.ds(0, _ROWS_PER_W * _C)],
                  out_hbm.at[pl.ds(lo * _C, _ROWS_PER_W * _C)])


def _segmax_call(a, src, dst):
  mesh = plsc.VectorSubcoreMesh(core_axis_name="c", subcore_axis_name="s")
  cp = pltpu.CompilerParams()
  if "needs_layout_passes" in pltpu.CompilerParams.__dataclass_fields__:
    cp = dataclasses.replace(cp, needs_layout_passes=False)
  kern = pl.kernel(
      _segmax_kernel,
      mesh=mesh,
      compiler_params=cp,
      out_type=jax.ShapeDtypeStruct((_NPAD * _C,), jnp.float32),
      scratch_types=[
          pltpu.VMEM((2 * _CHUNK,), jnp.int32),
          pltpu.VMEM((2 * _CHUNK,), jnp.int32),
          pltpu.VMEM((_WLCAP,), jnp.int32),
          pltpu.VMEM((_WLCAP,), jnp.int32),
          pltpu.VMEM((_GB, _C), jnp.float32),
          pltpu.VMEM((_GB, _C), jnp.float32),
          pltpu.VMEM(((_ROWS_PER_W + 8) * _C,), jnp.float32),
          pltpu.SemaphoreType.DMA,
          pltpu.SemaphoreType.DMA,
          pltpu.SemaphoreType.DMA,
      ],
  )
  return kern(a, src, dst)


def _out_body(s_ref, b_ref, x_ref, wg1_ref, bg1_ref, wg2_ref, bg2_ref, o_ref):
  s = s_ref[...]
  agg = _leaky(s + b_ref[...])
  agg = jnp.where(s < -1.0e38, 0.0, agg)
  g = _leaky(jnp.dot(agg, wg1_ref[...], preferred_element_type=jnp.float32)
             + bg1_ref[...])
  o_ref[...] = (x_ref[...]
                + jnp.dot(g, wg2_ref[...], preferred_element_type=jnp.float32)
                + bg2_ref[...])


def _out_call(s, b, x, wg1, bg12, wg2, bg22):
  n = x.shape[0]
  row = lambda i: (i, 0)
  fix = lambda i: (0, 0)
  return pl.pallas_call(
      _out_body,
      grid=(n // _BLK,),
      in_specs=[
          pl.BlockSpec((_BLK, _C), row),
          pl.BlockSpec((_BLK, _C), row),
          pl.BlockSpec((_BLK, _C), row),
          pl.BlockSpec((_C, _C), fix),
          pl.BlockSpec((1, _C), fix),
          pl.BlockSpec((_C, _C), fix),
          pl.BlockSpec((1, _C), fix),
      ],
      out_specs=pl.BlockSpec((_BLK, _C), row),
      out_shape=jax.ShapeDtypeStruct((n, _C), jnp.float32),
  )(s, b, x, wg1, bg12, wg2, bg22)


def kernel(x, pos, edge_index, Wh1, bh1, Wh2, bh2, Wf, bf, Wg1, bg1, Wg2, bg2):
  n = x.shape[0]
  e = edge_index.shape[1]

  xp = jnp.pad(x, ((0, _NPAD - n), (0, 0)))
  p8 = jnp.pad(pos, ((0, _NPAD - n), (0, 5)))
  wf38 = jnp.pad(Wf[:3], ((0, 5), (0, 0)))
  wfc = Wf[3:]
  wh28 = jnp.pad(Wh2, ((0, 0), (0, 5)))
  bh28 = jnp.pad(bh2, (0, 5)).reshape(1, 8)

  a, b = _ab_call(xp, p8, wfc, wf38, bf.reshape(1, _C), Wh1,
                  bh1.reshape(1, _C), wh28, bh28)

  src = edge_index[0].astype(jnp.int32)
  dst = edge_index[1].astype(jnp.int32)
  epad = (-e) % _CHUNK
  if epad:
    src = jnp.pad(src, (0, epad))
    dst = jnp.pad(dst, (0, epad), constant_values=_NPAD - 1)

  s_flat = _segmax_call(a, src, dst)
  s = s_flat.reshape(_NPAD, _C)

  out = _out_call(s, b, xp, Wg1, bg1.reshape(1, _C), Wg2, bg2.reshape(1, _C))
  return out[:n]


# speedup vs baseline: 4.5757x; 1.1272x over previous
import dataclasses
import functools

import jax
import jax.numpy as jnp
from jax import lax
from jax.experimental import pallas as pl
from jax.experimental.pallas import tpu as pltpu
from jax.experimental.pallas import tpu_sc as plsc

_NUM_CORES = 2
_NUM_SUBCORES = 16
_NW = _NUM_CORES * _NUM_SUBCORES
_LANES = 16

_C = 128
_W = _C // 2
_ROWS_PER_W = 320
_NPAD = _ROWS_PER_W * _NW
_CHUNK = 4000
_GB = 128
_WLCAP = _CHUNK + _GB + 16
_NEG = -3.0e38
_NEGBF2 = -8323200
_SHIFT = 14

_BLK = 1024


def _leaky(v):
  return jnp.where(v >= 0, v, 0.01 * v)


def _ab_body(x_ref, p8_ref, wfc_ref, wf38_ref, bf_ref, wh1_ref, bh1_ref,
             wh28_ref, bh28_ref, a_ref, b_ref):
  x = x_ref[...]
  p8 = p8_ref[...]
  h = _leaky(jnp.dot(x, wh1_ref[...], preferred_element_type=jnp.float32)
             + bh1_ref[...])
  d8 = jnp.tanh(jnp.dot(h, wh28_ref[...], preferred_element_type=jnp.float32)
                + bh28_ref[...])
  wf38 = wf38_ref[...]
  a_ref[...] = (jnp.dot(x, wfc_ref[...], preferred_element_type=jnp.float32)
                + jnp.dot(p8, wf38, preferred_element_type=jnp.float32)
                + bf_ref[...])
  b_ref[...] = jnp.dot(d8 - p8, wf38, preferred_element_type=jnp.float32)


def _ab_call(x, p8, wfc, wf38, bf2, wh1, bh12, wh28, bh28):
  n = x.shape[0]
  grid = (n // _BLK,)
  row = lambda i: (i, 0)
  fix = lambda i: (0, 0)
  return pl.pallas_call(
      _ab_body,
      grid=grid,
      in_specs=[
          pl.BlockSpec((_BLK, _C), row),
          pl.BlockSpec((_BLK, 8), row),
          pl.BlockSpec((_C, _C), fix),
          pl.BlockSpec((8, _C), fix),
          pl.BlockSpec((1, _C), fix),
          pl.BlockSpec((_C, _C), fix),
          pl.BlockSpec((1, _C), fix),
          pl.BlockSpec((_C, 8), fix),
          pl.BlockSpec((1, 8), fix),
      ],
      out_specs=[pl.BlockSpec((_BLK, _C), row), pl.BlockSpec((_BLK, _C), row)],
      out_shape=[jax.ShapeDtypeStruct((n, _C), jnp.float32)] * 2,
  )(x, p8, wfc, wf38, bf2, wh1, bh12, wh28, bh28)


def _segmax_kernel(a_hbm, ew_hbm, out_hbm,
                   ev, wls, wld, rbuf0, rbuf1, acc,
                   semc, semr0, semr1):
  wid = lax.axis_index("s") * _NUM_CORES + lax.axis_index("c")
  lo = wid * _ROWS_PER_W
  dummy = _ROWS_PER_W

  negp = jnp.full((_LANES,), _NEGBF2, jnp.int32)

  @pl.loop(0, (_ROWS_PER_W + 8) * _W, step=_LANES)
  def _(i):
    acc[pl.ds(i, _LANES)] = negp

  zeros = jnp.zeros((_LANES,), jnp.int32)
  dums = jnp.full((_LANES,), dummy, jnp.int32)

  @pl.loop(0, _WLCAP, step=_LANES)
  def _(i):
    wls[pl.ds(i, _LANES)] = zeros
    wld[pl.ds(i, _LANES)] = dums

  nchunks = ew_hbm.shape[0] // _CHUNK
  ngroups = _CHUNK // _LANES

  def fire_chunk(ci):
    off = lax.rem(ci, 2) * _CHUNK
    pltpu.async_copy(ew_hbm.at[pl.ds(ci * _CHUNK, _CHUNK)],
                     ev.at[pl.ds(off, _CHUNK)], semc)

  def wait_chunk():
    pltpu.make_async_copy(ew_hbm.at[pl.ds(0, _CHUNK)],
                          ev.at[pl.ds(0, _CHUNK)], semc).wait()

  def fire_batch(b, rbuf, semr):
    pltpu.async_copy(a_hbm.at[wls.at[pl.ds(b * _GB, _GB)]], rbuf, semr)

  def wait_batch(b, rbuf, semr):
    pltpu.make_async_copy(a_hbm.at[wls.at[pl.ds(b * _GB, _GB)]],
                          rbuf, semr).wait()

  def accum_batch(b, rbuf):
    @pl.loop(0, _GB // _LANES)
    def _(g):
      dvec = wld[pl.ds(b * _GB + g * _LANES, _LANES)]
      for l in range(_LANES):
        base = dvec[l] * _W
        j = g * _LANES + l
        for c in range(_W // _LANES):
          sl = pl.ds(base + c * _LANES, _LANES)
          av = plsc.bitcast(acc[sl], jnp.bfloat16)
          rv = plsc.bitcast(rbuf[j, pl.ds(c * _LANES, _LANES)], jnp.bfloat16)
          acc[sl] = plsc.bitcast(jnp.maximum(av, rv), jnp.int32)

  def scan_group(g, cnt, pbase):
    w16 = ev[pl.ds(pbase + g * _LANES, _LANES)]
    ld = lax.shift_right_logical(w16, _SHIFT) - lo
    mask = ld.astype(jnp.uint32) < jnp.uint32(_ROWS_PER_W)
    pop = plsc.all_reduce_population_count(mask)
    total = pop[0]

    @pl.when(total > 0)
    def _():
      mi = jnp.where(mask, 1, 0).astype(jnp.int32)
      pos = plsc.cumsum(mi) + (cnt - 1)
      s16 = w16 & ((1 << _SHIFT) - 1)
      plsc.store_scatter(wls, [pos], s16, mask=mask)
      plsc.store_scatter(wld, [pos], ld, mask=mask)

    return cnt + total

  def drain_all(cnt):
    nb = cnt // _GB

    @pl.when(nb > 0)
    def _():
      fire_batch(0, rbuf0, semr0)

    def body(b, carry):
      def even(x):
        @pl.when(b + 1 < nb)
        def _():
          fire_batch(b + 1, rbuf1, semr1)
        wait_batch(b, rbuf0, semr0)
        accum_batch(b, rbuf0)
        return x

      def odd(x):
        @pl.when(b + 1 < nb)
        def _():
          fire_batch(b + 1, rbuf0, semr0)
        wait_batch(b, rbuf1, semr1)
        accum_batch(b, rbuf1)
        return x

      return lax.cond(lax.rem(b, 2) == 0, even, odd, carry)

    lax.fori_loop(0, nb, body, jnp.int32(0))

    for k in range(_GB // _LANES):
      wls[pl.ds(k * _LANES, _LANES)] = wls[pl.ds(nb * _GB + k * _LANES,
                                                 _LANES)]
      wld[pl.ds(k * _LANES, _LANES)] = wld[pl.ds(nb * _GB + k * _LANES,
                                                 _LANES)]
    return cnt - nb * _GB

  fire_chunk(0)

  def chunk_body(ci, cnt):
    wait_chunk()

    @pl.when(ci + 1 < nchunks)
    def _():
      fire_chunk(ci + 1)

    pbase = lax.rem(ci, 2) * _CHUNK
    cnt = lax.fori_loop(0, ngroups, lambda g, c: scan_group(g, c, pbase), cnt)
    return drain_all(cnt)

  _ = lax.fori_loop(0, nchunks, chunk_body, jnp.int32(0))

  fire_batch(0, rbuf0, semr0)
  wait_batch(0, rbuf0, semr0)
  accum_batch(0, rbuf0)

  pltpu.sync_copy(acc.at[pl.ds(0, _ROWS_PER_W * _W)],
                  out_hbm.at[pl.ds(lo * _W, _ROWS_PER_W * _W)])


def _segmax_call(ap, ew):
  mesh = plsc.VectorSubcoreMesh(core_axis_name="c", subcore_axis_name="s")
  cp = pltpu.CompilerParams()
  if "needs_layout_passes" in pltpu.CompilerParams.__dataclass_fields__:
    cp = dataclasses.replace(cp, needs_layout_passes=False)
  if "use_tc_tiling_on_sc" in pltpu.CompilerParams.__dataclass_fields__:
    cp = dataclasses.replace(cp, use_tc_tiling_on_sc=False)
  kern = pl.kernel(
      _segmax_kernel,
      mesh=mesh,
      compiler_params=cp,
      out_type=jax.ShapeDtypeStruct((_NPAD * _W,), jnp.int32),
      scratch_types=[
          pltpu.VMEM((2 * _CHUNK,), jnp.int32),
          pltpu.VMEM((_WLCAP,), jnp.int32),
          pltpu.VMEM((_WLCAP,), jnp.int32),
          pltpu.VMEM((_GB, _W), jnp.int32),
          pltpu.VMEM((_GB, _W), jnp.int32),
          pltpu.VMEM(((_ROWS_PER_W + 8) * _W,), jnp.int32),
          pltpu.SemaphoreType.DMA,
          pltpu.SemaphoreType.DMA,
          pltpu.SemaphoreType.DMA,
      ],
  )
  return kern(ap, ew)


def _out_body(s_ref, b_ref, x_ref, wg1_ref, bg1_ref, wg2_ref, bg2_ref, o_ref):
  s = s_ref[...].astype(jnp.float32)
  agg = _leaky(s + b_ref[...])
  agg = jnp.where(s < -1.0e38, 0.0, agg)
  g = _leaky(jnp.dot(agg, wg1_ref[...], preferred_element_type=jnp.float32)
             + bg1_ref[...])
  o_ref[...] = (x_ref[...]
                + jnp.dot(g, wg2_ref[...], preferred_element_type=jnp.float32)
                + bg2_ref[...])


def _out_call(s, b, x, wg1, bg12, wg2, bg22):
  n = x.shape[0]
  row = lambda i: (i, 0)
  fix = lambda i: (0, 0)
  return pl.pallas_call(
      _out_body,
      grid=(n // _BLK,),
      in_specs=[
          pl.BlockSpec((_BLK, _C), row),
          pl.BlockSpec((_BLK, _C), row),
          pl.BlockSpec((_BLK, _C), row),
          pl.BlockSpec((_C, _C), fix),
          pl.BlockSpec((1, _C), fix),
          pl.BlockSpec((_C, _C), fix),
          pl.BlockSpec((1, _C), fix),
      ],
      out_specs=pl.BlockSpec((_BLK, _C), row),
      out_shape=jax.ShapeDtypeStruct((n, _C), jnp.float32),
  )(s, b, x, wg1, bg12, wg2, bg22)


def kernel(x, pos, edge_index, Wh1, bh1, Wh2, bh2, Wf, bf, Wg1, bg1, Wg2, bg2):
  n = x.shape[0]
  e = edge_index.shape[1]

  xp = jnp.pad(x, ((0, _NPAD - n), (0, 0)))
  p8 = jnp.pad(pos, ((0, _NPAD - n), (0, 5)))
  wf38 = jnp.pad(Wf[:3], ((0, 5), (0, 0)))
  wfc = Wf[3:]
  wh28 = jnp.pad(Wh2, ((0, 0), (0, 5)))
  bh28 = jnp.pad(bh2, (0, 5)).reshape(1, 8)

  a, b = _ab_call(xp, p8, wfc, wf38, bf.reshape(1, _C), Wh1,
                  bh1.reshape(1, _C), wh28, bh28)

  ap = lax.bitcast_convert_type(
      a.astype(jnp.bfloat16).reshape(_NPAD, _W, 2), jnp.int32)

  src = edge_index[0].astype(jnp.int32)
  dst = edge_index[1].astype(jnp.int32)
  ew = dst * (1 << _SHIFT) + src
  epad = (-e) % _CHUNK
  if epad:
    ew = jnp.pad(ew, (0, epad), constant_values=(_NPAD - 1) * (1 << _SHIFT))

  sp = _segmax_call(ap, ew)
  s = lax.bitcast_convert_type(
      sp.reshape(_NPAD, _W), jnp.bfloat16).reshape(_NPAD, _C)

  out = _out_call(s, b, xp, Wg1, bg1.reshape(1, _C), Wg2, bg2.reshape(1, _C))
  return out[:n]


# speedup vs baseline: 4.9408x; 1.0798x over previous
import dataclasses
import functools

import jax
import jax.numpy as jnp
from jax import lax
from jax.experimental import pallas as pl
from jax.experimental.pallas import tpu as pltpu
from jax.experimental.pallas import tpu_sc as plsc

_NUM_CORES = 2
_NUM_SUBCORES = 16
_NW = _NUM_CORES * _NUM_SUBCORES
_LANES = 16

_C = 128
_W = _C // 2
_ROWS_PER_W = 320
_NPAD = _ROWS_PER_W * _NW
_CHUNK = 4000
_GB = 128
_WLCAP = _CHUNK + _GB + 16
_NRB = 4
_NEG = -3.0e38
_NEGBF2 = -8323200
_SHIFT = 14

_BLK = 1024


def _leaky(v):
  return jnp.where(v >= 0, v, 0.01 * v)


def _ab_body(x_ref, p8_ref, wfc_ref, wf38_ref, bf_ref, wh1_ref, bh1_ref,
             wh28_ref, bh28_ref, a_ref, b_ref):
  x = x_ref[...]
  p8 = p8_ref[...]
  h = _leaky(jnp.dot(x, wh1_ref[...], preferred_element_type=jnp.float32)
             + bh1_ref[...])
  d8 = jnp.tanh(jnp.dot(h, wh28_ref[...], preferred_element_type=jnp.float32)
                + bh28_ref[...])
  wf38 = wf38_ref[...]
  a_ref[...] = (jnp.dot(x, wfc_ref[...], preferred_element_type=jnp.float32)
                + jnp.dot(p8, wf38, preferred_element_type=jnp.float32)
                + bf_ref[...])
  b_ref[...] = jnp.dot(d8 - p8, wf38, preferred_element_type=jnp.float32)


def _ab_call(x, p8, wfc, wf38, bf2, wh1, bh12, wh28, bh28):
  n = x.shape[0]
  grid = (n // _BLK,)
  row = lambda i: (i, 0)
  fix = lambda i: (0, 0)
  return pl.pallas_call(
      _ab_body,
      grid=grid,
      in_specs=[
          pl.BlockSpec((_BLK, _C), row),
          pl.BlockSpec((_BLK, 8), row),
          pl.BlockSpec((_C, _C), fix),
          pl.BlockSpec((8, _C), fix),
          pl.BlockSpec((1, _C), fix),
          pl.BlockSpec((_C, _C), fix),
          pl.BlockSpec((1, _C), fix),
          pl.BlockSpec((_C, 8), fix),
          pl.BlockSpec((1, 8), fix),
      ],
      out_specs=[pl.BlockSpec((_BLK, _C), row), pl.BlockSpec((_BLK, _C), row)],
      out_shape=[jax.ShapeDtypeStruct((n, _C), jnp.float32)] * 2,
  )(x, p8, wfc, wf38, bf2, wh1, bh12, wh28, bh28)


def _segmax_kernel(a_hbm, ew_hbm, out_hbm,
                   ev, wls, wld, rbuf0, rbuf1, rbuf2, rbuf3, rbufx, acc,
                   semc, semr0, semr1, semr2, semr3, semrx):
  wid = lax.axis_index("s") * _NUM_CORES + lax.axis_index("c")
  lo = wid * _ROWS_PER_W
  dummy = _ROWS_PER_W
  rbufs = (rbuf0, rbuf1, rbuf2, rbuf3)
  semrs = (semr0, semr1, semr2, semr3)

  negp = jnp.full((_LANES,), _NEGBF2, jnp.int32)

  @pl.loop(0, (_ROWS_PER_W + 8) * _W, step=_LANES)
  def _(i):
    acc[pl.ds(i, _LANES)] = negp

  zeros = jnp.zeros((_LANES,), jnp.int32)
  dums = jnp.full((_LANES,), dummy, jnp.int32)

  @pl.loop(0, 2 * _WLCAP, step=_LANES)
  def _(i):
    wls[pl.ds(i, _LANES)] = zeros
    wld[pl.ds(i, _LANES)] = dums

  nchunks = ew_hbm.shape[0] // _CHUNK
  ngroups = _CHUNK // _LANES

  def fire_chunk(ci):
    off = lax.rem(ci, 2) * _CHUNK
    pltpu.async_copy(ew_hbm.at[pl.ds(ci * _CHUNK, _CHUNK)],
                     ev.at[pl.ds(off, _CHUNK)], semc)

  def wait_chunk():
    pltpu.make_async_copy(ew_hbm.at[pl.ds(0, _CHUNK)],
                          ev.at[pl.ds(0, _CHUNK)], semc).wait()

  def fire_batch(woff, rbuf, semr):
    pltpu.async_copy(a_hbm.at[wls.at[pl.ds(woff, _GB)]], rbuf, semr)

  def wait_batch(woff, rbuf, semr):
    pltpu.make_async_copy(a_hbm.at[wls.at[pl.ds(woff, _GB)]],
                          rbuf, semr).wait()

  def accum_batch(woff, rbuf):
    @pl.loop(0, _GB // _LANES)
    def _(g):
      dvec = wld[pl.ds(woff + g * _LANES, _LANES)]
      for l in range(_LANES):
        base = dvec[l] * _W
        j = g * _LANES + l
        for c in range(_W // _LANES):
          sl = pl.ds(base + c * _LANES, _LANES)
          av = plsc.bitcast(acc[sl], jnp.bfloat16)
          rv = plsc.bitcast(rbuf[j, pl.ds(c * _LANES, _LANES)], jnp.bfloat16)
          acc[sl] = plsc.bitcast(jnp.maximum(av, rv), jnp.int32)

  def scan_group(g, cnt, pbase, pw):
    w16 = ev[pl.ds(pbase + g * _LANES, _LANES)]
    ld = lax.shift_right_logical(w16, _SHIFT) - lo
    mask = ld.astype(jnp.uint32) < jnp.uint32(_ROWS_PER_W)
    pop = plsc.all_reduce_population_count(mask)
    total = pop[0]

    @pl.when(total > 0)
    def _():
      mi = jnp.where(mask, 1, 0).astype(jnp.int32)
      pos = plsc.cumsum(mi) + (pw + cnt - 1)
      s16 = w16 & ((1 << _SHIFT) - 1)
      plsc.store_scatter(wls, [pos], s16, mask=mask)
      plsc.store_scatter(wld, [pos], ld, mask=mask)

    return cnt + total

  def accum_prev(nprev, pv):
    for r in range(_NRB):
      @pl.when(r < nprev)
      def _():
        wait_batch(pv + r * _GB, rbufs[r], semrs[r])
        accum_batch(pv + r * _GB, rbufs[r])

  fire_chunk(0)

  def chunk_body(ci, carry):
    cnt, nprev = carry
    pw = lax.rem(ci, 2) * _WLCAP
    pv = lax.rem(ci + 1, 2) * _WLCAP
    wait_chunk()

    @pl.when(ci + 1 < nchunks)
    def _():
      fire_chunk(ci + 1)

    pbase = lax.rem(ci, 2) * _CHUNK
    cnt = lax.fori_loop(
        0, ngroups, lambda g, c: scan_group(g, c, pbase, pw), cnt)

    nb = cnt // _GB
    nfire = jnp.minimum(nb, _NRB)

    accum_prev(nprev, pv)

    for r in range(_NRB):
      @pl.when(r < nfire)
      def _():
        fire_batch(pw + r * _GB, rbufs[r], semrs[r])

    def extra(b, carry2):
      fire_batch(pw + b * _GB, rbufx, semrx)
      wait_batch(pw + b * _GB, rbufx, semrx)
      accum_batch(pw + b * _GB, rbufx)
      return carry2

    lax.fori_loop(_NRB, nb, extra, jnp.int32(0))

    for k in range(_GB // _LANES):
      wls[pl.ds(pv + k * _LANES, _LANES)] = wls[pl.ds(
          pw + nb * _GB + k * _LANES, _LANES)]
      wld[pl.ds(pv + k * _LANES, _LANES)] = wld[pl.ds(
          pw + nb * _GB + k * _LANES, _LANES)]
    return cnt - nb * _GB, nfire

  cnt, nprev = lax.fori_loop(0, nchunks, chunk_body,
                             (jnp.int32(0), jnp.int32(0)))

  p_end = lax.rem(nchunks, 2) * _WLCAP
  p_last = lax.rem(nchunks + 1, 2) * _WLCAP
  accum_prev(nprev, p_last)
  fire_batch(p_end, rbufx, semrx)
  wait_batch(p_end, rbufx, semrx)
  accum_batch(p_end, rbufx)

  pltpu.sync_copy(acc.at[pl.ds(0, _ROWS_PER_W * _W)],
                  out_hbm.at[pl.ds(lo * _W, _ROWS_PER_W * _W)])


def _segmax_call(ap, ew):
  mesh = plsc.VectorSubcoreMesh(core_axis_name="c", subcore_axis_name="s")
  cp = pltpu.CompilerParams()
  if "needs_layout_passes" in pltpu.CompilerParams.__dataclass_fields__:
    cp = dataclasses.replace(cp, needs_layout_passes=False)
  if "use_tc_tiling_on_sc" in pltpu.CompilerParams.__dataclass_fields__:
    cp = dataclasses.replace(cp, use_tc_tiling_on_sc=False)
  kern = pl.kernel(
      _segmax_kernel,
      mesh=mesh,
      compiler_params=cp,
      out_type=jax.ShapeDtypeStruct((_NPAD * _W,), jnp.int32),
      scratch_types=[
          pltpu.VMEM((2 * _CHUNK,), jnp.int32),
          pltpu.VMEM((2 * _WLCAP,), jnp.int32),
          pltpu.VMEM((2 * _WLCAP,), jnp.int32),
          pltpu.VMEM((_GB, _W), jnp.int32),
          pltpu.VMEM((_GB, _W), jnp.int32),
          pltpu.VMEM((_GB, _W), jnp.int32),
          pltpu.VMEM((_GB, _W), jnp.int32),
          pltpu.VMEM((_GB, _W), jnp.int32),
          pltpu.VMEM(((_ROWS_PER_W + 8) * _W,), jnp.int32),
          pltpu.SemaphoreType.DMA,
          pltpu.SemaphoreType.DMA,
          pltpu.SemaphoreType.DMA,
          pltpu.SemaphoreType.DMA,
          pltpu.SemaphoreType.DMA,
          pltpu.SemaphoreType.DMA,
      ],
  )
  return kern(ap, ew)


def _out_body(s_ref, b_ref, x_ref, wg1_ref, bg1_ref, wg2_ref, bg2_ref, o_ref):
  s = s_ref[...].astype(jnp.float32)
  agg = _leaky(s + b_ref[...])
  agg = jnp.where(s < -1.0e38, 0.0, agg)
  g = _leaky(jnp.dot(agg, wg1_ref[...], preferred_element_type=jnp.float32)
             + bg1_ref[...])
  o_ref[...] = (x_ref[...]
                + jnp.dot(g, wg2_ref[...], preferred_element_type=jnp.float32)
                + bg2_ref[...])


def _out_call(s, b, x, wg1, bg12, wg2, bg22):
  n = x.shape[0]
  row = lambda i: (i, 0)
  fix = lambda i: (0, 0)
  return pl.pallas_call(
      _out_body,
      grid=(n // _BLK,),
      in_specs=[
          pl.BlockSpec((_BLK, _C), row),
          pl.BlockSpec((_BLK, _C), row),
          pl.BlockSpec((_BLK, _C), row),
          pl.BlockSpec((_C, _C), fix),
          pl.BlockSpec((1, _C), fix),
          pl.BlockSpec((_C, _C), fix),
          pl.BlockSpec((1, _C), fix),
      ],
      out_specs=pl.BlockSpec((_BLK, _C), row),
      out_shape=jax.ShapeDtypeStruct((n, _C), jnp.float32),
  )(s, b, x, wg1, bg12, wg2, bg22)


def kernel(x, pos, edge_index, Wh1, bh1, Wh2, bh2, Wf, bf, Wg1, bg1, Wg2, bg2):
  n = x.shape[0]
  e = edge_index.shape[1]

  xp = jnp.pad(x, ((0, _NPAD - n), (0, 0)))
  p8 = jnp.pad(pos, ((0, _NPAD - n), (0, 5)))
  wf38 = jnp.pad(Wf[:3], ((0, 5), (0, 0)))
  wfc = Wf[3:]
  wh28 = jnp.pad(Wh2, ((0, 0), (0, 5)))
  bh28 = jnp.pad(bh2, (0, 5)).reshape(1, 8)

  a, b = _ab_call(xp, p8, wfc, wf38, bf.reshape(1, _C), Wh1,
                  bh1.reshape(1, _C), wh28, bh28)

  ap = lax.bitcast_convert_type(
      a.astype(jnp.bfloat16).reshape(_NPAD, _W, 2), jnp.int32)

  src = edge_index[0].astype(jnp.int32)
  dst = edge_index[1].astype(jnp.int32)
  ew = dst * (1 << _SHIFT) + src
  epad = (-e) % _CHUNK
  if epad:
    ew = jnp.pad(ew, (0, epad), constant_values=(_NPAD - 1) * (1 << _SHIFT))

  sp = _segmax_call(ap, ew)
  s = lax.bitcast_convert_type(
      sp.reshape(_NPAD, _W), jnp.bfloat16).reshape(_NPAD, _C)

  out = _out_call(s, b, xp, Wg1, bg1.reshape(1, _C), Wg2, bg2.reshape(1, _C))
  return out[:n]


# speedup vs baseline: 5.0657x; 1.0253x over previous
import dataclasses
import functools

import jax
import jax.numpy as jnp
from jax import lax
from jax.experimental import pallas as pl
from jax.experimental.pallas import tpu as pltpu
from jax.experimental.pallas import tpu_sc as plsc

_NUM_CORES = 2
_NUM_SUBCORES = 16
_NW = _NUM_CORES * _NUM_SUBCORES
_LANES = 16

_C = 128
_W = _C // 2
_ROWS_PER_W = 320
_NPAD = _ROWS_PER_W * _NW
_CHUNK = 4096
_NQ = 4
_QE = _CHUNK // _NQ
_GB = 128
_QCAP = _QE + _GB + 16
_WLHALF = _NQ * _QCAP
_NEG = -3.0e38
_NEGBF2 = -8323200
_SHIFT = 14

_BLK = 1024


def _leaky(v):
  return jnp.where(v >= 0, v, 0.01 * v)


def _ab_body(x_ref, p8_ref, wfc_ref, wf38_ref, bf_ref, wh1_ref, bh1_ref,
             wh28_ref, bh28_ref, a_ref, b_ref):
  x = x_ref[...]
  p8 = p8_ref[...]
  h = _leaky(jnp.dot(x, wh1_ref[...], preferred_element_type=jnp.float32)
             + bh1_ref[...])
  d8 = jnp.tanh(jnp.dot(h, wh28_ref[...], preferred_element_type=jnp.float32)
                + bh28_ref[...])
  wf38 = wf38_ref[...]
  a_ref[...] = (jnp.dot(x, wfc_ref[...], preferred_element_type=jnp.float32)
                + jnp.dot(p8, wf38, preferred_element_type=jnp.float32)
                + bf_ref[...])
  b_ref[...] = jnp.dot(d8 - p8, wf38, preferred_element_type=jnp.float32)


def _ab_call(x, p8, wfc, wf38, bf2, wh1, bh12, wh28, bh28):
  n = x.shape[0]
  grid = (n // _BLK,)
  row = lambda i: (i, 0)
  fix = lambda i: (0, 0)
  return pl.pallas_call(
      _ab_body,
      grid=grid,
      in_specs=[
          pl.BlockSpec((_BLK, _C), row),
          pl.BlockSpec((_BLK, 8), row),
          pl.BlockSpec((_C, _C), fix),
          pl.BlockSpec((8, _C), fix),
          pl.BlockSpec((1, _C), fix),
          pl.BlockSpec((_C, _C), fix),
          pl.BlockSpec((1, _C), fix),
          pl.BlockSpec((_C, 8), fix),
          pl.BlockSpec((1, 8), fix),
      ],
      out_specs=[pl.BlockSpec((_BLK, _C), row), pl.BlockSpec((_BLK, _C), row)],
      out_shape=[jax.ShapeDtypeStruct((n, _C), jnp.float32)] * 2,
  )(x, p8, wfc, wf38, bf2, wh1, bh12, wh28, bh28)


def _segmax_kernel(a_hbm, ew_hbm, out_hbm,
                   ev, wl, st0, st1, st2, st3, stx,
                   rbuf0, rbuf1, rbuf2, rbuf3, rbufx, acc,
                   semc, semr0, semr1, semr2, semr3, semrx):
  wid = lax.axis_index("s") * _NUM_CORES + lax.axis_index("c")
  lo = wid * _ROWS_PER_W
  losh = lo * (1 << _SHIFT)
  rbufs = (rbuf0, rbuf1, rbuf2, rbuf3)
  semrs = (semr0, semr1, semr2, semr3)
  stages = (st0, st1, st2, st3)

  negp = jnp.full((_LANES,), _NEGBF2, jnp.int32)

  @pl.loop(0, (_ROWS_PER_W + 8) * _W, step=_LANES)
  def _(i):
    acc[pl.ds(i, _LANES)] = negp

  dums = jnp.full((_LANES,), _ROWS_PER_W * (1 << _SHIFT), jnp.int32)

  @pl.loop(0, 2 * _WLHALF, step=_LANES)
  def _(i):
    wl[pl.ds(i, _LANES)] = dums

  nchunks = ew_hbm.shape[0] // _CHUNK
  kiters = _QE // _LANES

  def fire_chunk(ci):
    off = lax.rem(ci, 2) * _CHUNK
    pltpu.async_copy(ew_hbm.at[pl.ds(ci * _CHUNK, _CHUNK)],
                     ev.at[pl.ds(off, _CHUNK)], semc)

  def wait_chunk():
    pltpu.make_async_copy(ew_hbm.at[pl.ds(0, _CHUNK)],
                          ev.at[pl.ds(0, _CHUNK)], semc).wait()

  def fire_batch(woff, stage, rbuf, semr):
    for k in range(_GB // _LANES):
      stage[pl.ds(k * _LANES, _LANES)] = (
          wl[pl.ds(woff + k * _LANES, _LANES)] & ((1 << _SHIFT) - 1))
    pltpu.async_copy(a_hbm.at[stage], rbuf, semr)

  def wait_batch(stage, rbuf, semr):
    pltpu.make_async_copy(a_hbm.at[stage], rbuf, semr).wait()

  def accum_batch(woff, rbuf):
    @pl.loop(0, _GB // _LANES)
    def _(g):
      dvec = lax.shift_right_logical(
          wl[pl.ds(woff + g * _LANES, _LANES)], _SHIFT)
      for l in range(_LANES):
        base = dvec[l] * _W
        j = g * _LANES + l
        for c in range(_W // _LANES):
          sl = pl.ds(base + c * _LANES, _LANES)
          av = plsc.bitcast(acc[sl], jnp.bfloat16)
          rv = plsc.bitcast(rbuf[j, pl.ds(c * _LANES, _LANES)], jnp.bfloat16)
          acc[sl] = plsc.bitcast(jnp.maximum(av, rv), jnp.int32)

  fire_chunk(0)

  thresh = jnp.uint32(_ROWS_PER_W * (1 << _SHIFT))

  def chunk_body(ci, carry):
    cnts, nfs = carry
    pw = lax.rem(ci, 2) * _WLHALF
    pv = lax.rem(ci + 1, 2) * _WLHALF
    wait_chunk()

    @pl.when(ci + 1 < nchunks)
    def _():
      fire_chunk(ci + 1)

    pbase = lax.rem(ci, 2) * _CHUNK

    def scan_iter(k, cs):
      out = []
      for q in range(_NQ):
        w16 = ev[pl.ds(pbase + q * _QE + k * _LANES, _LANES)]
        t = w16 - losh
        mask = t.astype(jnp.uint32) < thresh
        mi = mask.astype(jnp.int32)
        csum = plsc.cumsum(mi)
        pos = csum + (pw + q * _QCAP + cs[q] - 1)
        plsc.store_scatter(wl, [pos], t, mask=mask)
        out.append(cs[q] + csum[_LANES - 1])
      return tuple(out)

    cnts = lax.fori_loop(0, kiters, scan_iter, cnts)

    new_cnts, new_nfs = [], []
    for q in range(_NQ):
      pwq = pw + q * _QCAP
      pvq = pv + q * _QCAP
      nb = cnts[q] // _GB

      @pl.when(nfs[q] > 0)
      def _():
        wait_batch(stages[q], rbufs[q], semrs[q])
        accum_batch(pvq, rbufs[q])

      @pl.when(nb > 0)
      def _():
        fire_batch(pwq, stages[q], rbufs[q], semrs[q])

      def extra(b, cc):
        fire_batch(pwq + b * _GB, stx, rbufx, semrx)
        wait_batch(stx, rbufx, semrx)
        accum_batch(pwq + b * _GB, rbufx)
        return cc

      lax.fori_loop(1, nb, extra, jnp.int32(0))

      for k in range(_GB // _LANES):
        wl[pl.ds(pvq + k * _LANES, _LANES)] = wl[pl.ds(
            pwq + nb * _GB + k * _LANES, _LANES)]
      new_cnts.append(cnts[q] - nb * _GB)
      new_nfs.append(jnp.minimum(nb, 1))

    return tuple(new_cnts), tuple(new_nfs)

  zero4 = (jnp.int32(0),) * _NQ
  cnts, nfs = lax.fori_loop(0, nchunks, chunk_body, (zero4, zero4))

  p_end = lax.rem(nchunks, 2) * _WLHALF
  p_last = lax.rem(nchunks + 1, 2) * _WLHALF
  for q in range(_NQ):
    @pl.when(nfs[q] > 0)
    def _():
      wait_batch(stages[q], rbufs[q], semrs[q])
      accum_batch(p_last + q * _QCAP, rbufs[q])
  for q in range(_NQ):
    fire_batch(p_end + q * _QCAP, stx, rbufx, semrx)
    wait_batch(stx, rbufx, semrx)
    accum_batch(p_end + q * _QCAP, rbufx)

  pltpu.sync_copy(acc.at[pl.ds(0, _ROWS_PER_W * _W)],
                  out_hbm.at[pl.ds(lo * _W, _ROWS_PER_W * _W)])


def _segmax_call(ap, ew):
  mesh = plsc.VectorSubcoreMesh(core_axis_name="c", subcore_axis_name="s")
  cp = pltpu.CompilerParams()
  if "needs_layout_passes" in pltpu.CompilerParams.__dataclass_fields__:
    cp = dataclasses.replace(cp, needs_layout_passes=False)
  if "use_tc_tiling_on_sc" in pltpu.CompilerParams.__dataclass_fields__:
    cp = dataclasses.replace(cp, use_tc_tiling_on_sc=False)
  kern = pl.kernel(
      _segmax_kernel,
      mesh=mesh,
      compiler_params=cp,
      out_type=jax.ShapeDtypeStruct((_NPAD * _W,), jnp.int32),
      scratch_types=[
          pltpu.VMEM((2 * _CHUNK,), jnp.int32),
          pltpu.VMEM((2 * _WLHALF,), jnp.int32),
          pltpu.VMEM((_GB,), jnp.int32),
          pltpu.VMEM((_GB,), jnp.int32),
          pltpu.VMEM((_GB,), jnp.int32),
          pltpu.VMEM((_GB,), jnp.int32),
          pltpu.VMEM((_GB,), jnp.int32),
          pltpu.VMEM((_GB, _W), jnp.int32),
          pltpu.VMEM((_GB, _W), jnp.int32),
          pltpu.VMEM((_GB, _W), jnp.int32),
          pltpu.VMEM((_GB, _W), jnp.int32),
          pltpu.VMEM((_GB, _W), jnp.int32),
          pltpu.VMEM(((_ROWS_PER_W + 8) * _W,), jnp.int32),
          pltpu.SemaphoreType.DMA,
          pltpu.SemaphoreType.DMA,
          pltpu.SemaphoreType.DMA,
          pltpu.SemaphoreType.DMA,
          pltpu.SemaphoreType.DMA,
          pltpu.SemaphoreType.DMA,
      ],
  )
  return kern(ap, ew)


def _out_body(s_ref, b_ref, x_ref, wg1_ref, bg1_ref, wg2_ref, bg2_ref, o_ref):
  s = s_ref[...].astype(jnp.float32)
  agg = _leaky(s + b_ref[...])
  agg = jnp.where(s < -1.0e38, 0.0, agg)
  g = _leaky(jnp.dot(agg, wg1_ref[...], preferred_element_type=jnp.float32)
             + bg1_ref[...])
  o_ref[...] = (x_ref[...]
                + jnp.dot(g, wg2_ref[...], preferred_element_type=jnp.float32)
                + bg2_ref[...])


def _out_call(s, b, x, wg1, bg12, wg2, bg22):
  n = x.shape[0]
  row = lambda i: (i, 0)
  fix = lambda i: (0, 0)
  return pl.pallas_call(
      _out_body,
      grid=(n // _BLK,),
      in_specs=[
          pl.BlockSpec((_BLK, _C), row),
          pl.BlockSpec((_BLK, _C), row),
          pl.BlockSpec((_BLK, _C), row),
          pl.BlockSpec((_C, _C), fix),
          pl.BlockSpec((1, _C), fix),
          pl.BlockSpec((_C, _C), fix),
          pl.BlockSpec((1, _C), fix),
      ],
      out_specs=pl.BlockSpec((_BLK, _C), row),
      out_shape=jax.ShapeDtypeStruct((n, _C), jnp.float32),
  )(s, b, x, wg1, bg12, wg2, bg22)


def kernel(x, pos, edge_index, Wh1, bh1, Wh2, bh2, Wf, bf, Wg1, bg1, Wg2, bg2):
  n = x.shape[0]
  e = edge_index.shape[1]

  xp = jnp.pad(x, ((0, _NPAD - n), (0, 0)))
  p8 = jnp.pad(pos, ((0, _NPAD - n), (0, 5)))
  wf38 = jnp.pad(Wf[:3], ((0, 5), (0, 0)))
  wfc = Wf[3:]
  wh28 = jnp.pad(Wh2, ((0, 0), (0, 5)))
  bh28 = jnp.pad(bh2, (0, 5)).reshape(1, 8)

  a, b = _ab_call(xp, p8, wfc, wf38, bf.reshape(1, _C), Wh1,
                  bh1.reshape(1, _C), wh28, bh28)

  ap = lax.bitcast_convert_type(
      a.astype(jnp.bfloat16).reshape(_NPAD, _W, 2), jnp.int32)

  src = edge_index[0].astype(jnp.int32)
  dst = edge_index[1].astype(jnp.int32)
  ew = dst * (1 << _SHIFT) + src
  epad = (-e) % _CHUNK
  if epad:
    ew = jnp.pad(ew, (0, epad), constant_values=(_NPAD - 1) * (1 << _SHIFT))

  sp = _segmax_call(ap, ew)
  s = lax.bitcast_convert_type(
      sp.reshape(_NPAD, _W), jnp.bfloat16).reshape(_NPAD, _C)

  out = _out_call(s, b, xp, Wg1, bg1.reshape(1, _C), Wg2, bg2.reshape(1, _C))
  return out[:n]


# speedup vs baseline: 5.3040x; 1.0470x over previous
import dataclasses
import functools

import jax
import jax.numpy as jnp
from jax import lax
from jax.experimental import pallas as pl
from jax.experimental.pallas import tpu as pltpu
from jax.experimental.pallas import tpu_sc as plsc

_NUM_CORES = 2
_NUM_SUBCORES = 16
_NW = _NUM_CORES * _NUM_SUBCORES
_LANES = 16

_C = 128
_W = _C // 2
_ROWS_PER_W = 320
_NPAD = _ROWS_PER_W * _NW
_CHUNK = 4096
_NQ = 4
_QE = _CHUNK // _NQ
_GB = 128
_QCAP = _QE + _GB + 16
_WLHALF = _NQ * _QCAP
_NEG = -3.0e38
_NEGBF2 = -8323200
_SHIFT = 14

_BLK = 1024


def _leaky(v):
  return jnp.where(v >= 0, v, 0.01 * v)


def _ab_body(x_ref, p8_ref, wfc_ref, wf38_ref, bf_ref, wh1_ref, bh1_ref,
             wh28_ref, bh28_ref, a_ref, b_ref):
  x = x_ref[...]
  p8 = p8_ref[...]
  h = _leaky(jnp.dot(x, wh1_ref[...], preferred_element_type=jnp.float32)
             + bh1_ref[...])
  d8 = jnp.tanh(jnp.dot(h, wh28_ref[...], preferred_element_type=jnp.float32)
                + bh28_ref[...])
  wf38 = wf38_ref[...]
  a_ref[...] = (jnp.dot(x, wfc_ref[...], preferred_element_type=jnp.float32)
                + jnp.dot(p8, wf38, preferred_element_type=jnp.float32)
                + bf_ref[...])
  b_ref[...] = jnp.dot(d8 - p8, wf38, preferred_element_type=jnp.float32)


def _ab_call(x, p8, wfc, wf38, bf2, wh1, bh12, wh28, bh28):
  n = x.shape[0]
  grid = (n // _BLK,)
  row = lambda i: (i, 0)
  fix = lambda i: (0, 0)
  return pl.pallas_call(
      _ab_body,
      grid=grid,
      in_specs=[
          pl.BlockSpec((_BLK, _C), row),
          pl.BlockSpec((_BLK, 8), row),
          pl.BlockSpec((_C, _C), fix),
          pl.BlockSpec((8, _C), fix),
          pl.BlockSpec((1, _C), fix),
          pl.BlockSpec((_C, _C), fix),
          pl.BlockSpec((1, _C), fix),
          pl.BlockSpec((_C, 8), fix),
          pl.BlockSpec((1, 8), fix),
      ],
      out_specs=[pl.BlockSpec((_BLK, _C), row), pl.BlockSpec((_BLK, _C), row)],
      out_shape=[jax.ShapeDtypeStruct((n, _C), jnp.float32)] * 2,
  )(x, p8, wfc, wf38, bf2, wh1, bh12, wh28, bh28)


def _segmax_kernel(a_hbm, ew_hbm, out_hbm,
                   ev, wl, st0, st1, st2, st3, stx,
                   rbuf0, rbuf1, rbuf2, rbuf3, rbufx, acc,
                   semc, semr0, semr1, semr2, semr3, semrx):
  wid = lax.axis_index("s") * _NUM_CORES + lax.axis_index("c")
  lo = wid * _ROWS_PER_W
  losh = lo * (1 << _SHIFT)
  rbufs = (rbuf0, rbuf1, rbuf2, rbuf3)
  semrs = (semr0, semr1, semr2, semr3)
  stages = (st0, st1, st2, st3)

  negp = jnp.full((_LANES,), _NEGBF2, jnp.int32)

  @pl.loop(0, (_ROWS_PER_W + 8) * _W, step=_LANES)
  def _(i):
    acc[pl.ds(i, _LANES)] = negp

  dums = jnp.full((_LANES,), _ROWS_PER_W * (1 << _SHIFT), jnp.int32)

  @pl.loop(0, 2 * _WLHALF, step=_LANES)
  def _(i):
    wl[pl.ds(i, _LANES)] = dums

  nchunks = ew_hbm.shape[0] // _CHUNK
  kiters = _QE // _LANES

  def fire_chunk(ci):
    off = lax.rem(ci, 2) * _CHUNK
    pltpu.async_copy(ew_hbm.at[pl.ds(ci * _CHUNK, _CHUNK)],
                     ev.at[pl.ds(off, _CHUNK)], semc)

  def wait_chunk():
    pltpu.make_async_copy(ew_hbm.at[pl.ds(0, _CHUNK)],
                          ev.at[pl.ds(0, _CHUNK)], semc).wait()

  def fire_batch(woff, stage, rbuf, semr):
    for k in range(_GB // _LANES):
      stage[pl.ds(k * _LANES, _LANES)] = (
          wl[pl.ds(woff + k * _LANES, _LANES)] & ((1 << _SHIFT) - 1))
    pltpu.async_copy(a_hbm.at[stage], rbuf, semr)

  def wait_batch(stage, rbuf, semr):
    pltpu.make_async_copy(a_hbm.at[stage], rbuf, semr).wait()

  def accum_batch(woff, rbuf):
    @pl.loop(0, _GB // _LANES)
    def _(g):
      dvec = lax.shift_right_logical(
          wl[pl.ds(woff + g * _LANES, _LANES)], _SHIFT)
      for l in range(_LANES):
        base = dvec[l] * _W
        j = g * _LANES + l
        for c in range(_W // _LANES):
          sl = pl.ds(base + c * _LANES, _LANES)
          av = plsc.bitcast(acc[sl], jnp.bfloat16)
          rv = plsc.bitcast(rbuf[j, pl.ds(c * _LANES, _LANES)], jnp.bfloat16)
          acc[sl] = plsc.bitcast(jnp.maximum(av, rv), jnp.int32)

  fire_chunk(0)

  thresh = jnp.uint32(_ROWS_PER_W * (1 << _SHIFT))

  def chunk_body(ci, carry):
    cnts, nfs = carry
    pw = lax.rem(ci, 2) * _WLHALF
    pv = lax.rem(ci + 1, 2) * _WLHALF
    wait_chunk()

    @pl.when(ci + 1 < nchunks)
    def _():
      fire_chunk(ci + 1)

    pbase = lax.rem(ci, 2) * _CHUNK

    def scan_iter(k, cvs):
      out = []
      for q in range(_NQ):
        w16 = ev[pl.ds(pbase + q * _QE + k * _LANES, _LANES)]
        t = w16 - losh
        mask = t.astype(jnp.uint32) < thresh
        mi = mask.astype(jnp.int32)
        csum = plsc.cumsum(mi)
        pos = csum + cvs[q]
        plsc.store_scatter(wl, [pos], t, mask=mask)
        out.append(cvs[q] + plsc.all_reduce_population_count(mask))
      return tuple(out)

    bases = [pw + q * _QCAP for q in range(_NQ)]
    cvs0 = tuple(
        jnp.full((_LANES,), 1, jnp.int32) * (bases[q] + cnts[q] - 1)
        for q in range(_NQ))
    cvs = lax.fori_loop(0, kiters, scan_iter, cvs0)
    cnts = tuple(cvs[q][0] - (bases[q] - 1) for q in range(_NQ))

    new_cnts, new_nfs = [], []
    for q in range(_NQ):
      pwq = pw + q * _QCAP
      pvq = pv + q * _QCAP
      nb = cnts[q] // _GB

      @pl.when(nfs[q] > 0)
      def _():
        wait_batch(stages[q], rbufs[q], semrs[q])
        accum_batch(pvq, rbufs[q])

      @pl.when(nb > 0)
      def _():
        fire_batch(pwq, stages[q], rbufs[q], semrs[q])

      def extra(b, cc):
        fire_batch(pwq + b * _GB, stx, rbufx, semrx)
        wait_batch(stx, rbufx, semrx)
        accum_batch(pwq + b * _GB, rbufx)
        return cc

      lax.fori_loop(1, nb, extra, jnp.int32(0))

      for k in range(_GB // _LANES):
        wl[pl.ds(pvq + k * _LANES, _LANES)] = wl[pl.ds(
            pwq + nb * _GB + k * _LANES, _LANES)]
      new_cnts.append(cnts[q] - nb * _GB)
      new_nfs.append(jnp.minimum(nb, 1))

    return tuple(new_cnts), tuple(new_nfs)

  zero4 = (jnp.int32(0),) * _NQ
  cnts, nfs = lax.fori_loop(0, nchunks, chunk_body, (zero4, zero4))

  p_end = lax.rem(nchunks, 2) * _WLHALF
  p_last = lax.rem(nchunks + 1, 2) * _WLHALF
  for q in range(_NQ):
    @pl.when(nfs[q] > 0)
    def _():
      wait_batch(stages[q], rbufs[q], semrs[q])
      accum_batch(p_last + q * _QCAP, rbufs[q])
  for q in range(_NQ):
    fire_batch(p_end + q * _QCAP, stx, rbufx, semrx)
    wait_batch(stx, rbufx, semrx)
    accum_batch(p_end + q * _QCAP, rbufx)

  pltpu.sync_copy(acc.at[pl.ds(0, _ROWS_PER_W * _W)],
                  out_hbm.at[pl.ds(lo * _W, _ROWS_PER_W * _W)])


def _segmax_call(ap, ew):
  mesh = plsc.VectorSubcoreMesh(core_axis_name="c", subcore_axis_name="s")
  cp = pltpu.CompilerParams()
  if "needs_layout_passes" in pltpu.CompilerParams.__dataclass_fields__:
    cp = dataclasses.replace(cp, needs_layout_passes=False)
  if "use_tc_tiling_on_sc" in pltpu.CompilerParams.__dataclass_fields__:
    cp = dataclasses.replace(cp, use_tc_tiling_on_sc=False)
  kern = pl.kernel(
      _segmax_kernel,
      mesh=mesh,
      compiler_params=cp,
      out_type=jax.ShapeDtypeStruct((_NPAD * _W,), jnp.int32),
      scratch_types=[
          pltpu.VMEM((2 * _CHUNK,), jnp.int32),
          pltpu.VMEM((2 * _WLHALF,), jnp.int32),
          pltpu.VMEM((_GB,), jnp.int32),
          pltpu.VMEM((_GB,), jnp.int32),
          pltpu.VMEM((_GB,), jnp.int32),
          pltpu.VMEM((_GB,), jnp.int32),
          pltpu.VMEM((_GB,), jnp.int32),
          pltpu.VMEM((_GB, _W), jnp.int32),
          pltpu.VMEM((_GB, _W), jnp.int32),
          pltpu.VMEM((_GB, _W), jnp.int32),
          pltpu.VMEM((_GB, _W), jnp.int32),
          pltpu.VMEM((_GB, _W), jnp.int32),
          pltpu.VMEM(((_ROWS_PER_W + 8) * _W,), jnp.int32),
          pltpu.SemaphoreType.DMA,
          pltpu.SemaphoreType.DMA,
          pltpu.SemaphoreType.DMA,
          pltpu.SemaphoreType.DMA,
          pltpu.SemaphoreType.DMA,
          pltpu.SemaphoreType.DMA,
      ],
  )
  return kern(ap, ew)


def _out_body(s_ref, b_ref, x_ref, wg1_ref, bg1_ref, wg2_ref, bg2_ref, o_ref):
  s = s_ref[...].astype(jnp.float32)
  agg = _leaky(s + b_ref[...])
  agg = jnp.where(s < -1.0e38, 0.0, agg)
  g = _leaky(jnp.dot(agg, wg1_ref[...], preferred_element_type=jnp.float32)
             + bg1_ref[...])
  o_ref[...] = (x_ref[...]
                + jnp.dot(g, wg2_ref[...], preferred_element_type=jnp.float32)
                + bg2_ref[...])


def _out_call(s, b, x, wg1, bg12, wg2, bg22):
  n = x.shape[0]
  row = lambda i: (i, 0)
  fix = lambda i: (0, 0)
  return pl.pallas_call(
      _out_body,
      grid=(n // _BLK,),
      in_specs=[
          pl.BlockSpec((_BLK, _C), row),
          pl.BlockSpec((_BLK, _C), row),
          pl.BlockSpec((_BLK, _C), row),
          pl.BlockSpec((_C, _C), fix),
          pl.BlockSpec((1, _C), fix),
          pl.BlockSpec((_C, _C), fix),
          pl.BlockSpec((1, _C), fix),
      ],
      out_specs=pl.BlockSpec((_BLK, _C), row),
      out_shape=jax.ShapeDtypeStruct((n, _C), jnp.float32),
  )(s, b, x, wg1, bg12, wg2, bg22)


def kernel(x, pos, edge_index, Wh1, bh1, Wh2, bh2, Wf, bf, Wg1, bg1, Wg2, bg2):
  n = x.shape[0]
  e = edge_index.shape[1]

  xp = jnp.pad(x, ((0, _NPAD - n), (0, 0)))
  p8 = jnp.pad(pos, ((0, _NPAD - n), (0, 5)))
  wf38 = jnp.pad(Wf[:3], ((0, 5), (0, 0)))
  wfc = Wf[3:]
  wh28 = jnp.pad(Wh2, ((0, 0), (0, 5)))
  bh28 = jnp.pad(bh2, (0, 5)).reshape(1, 8)

  a, b = _ab_call(xp, p8, wfc, wf38, bf.reshape(1, _C), Wh1,
                  bh1.reshape(1, _C), wh28, bh28)

  ap = lax.bitcast_convert_type(
      a.astype(jnp.bfloat16).reshape(_NPAD, _W, 2), jnp.int32)

  src = edge_index[0].astype(jnp.int32)
  dst = edge_index[1].astype(jnp.int32)
  ew = dst * (1 << _SHIFT) + src
  epad = (-e) % _CHUNK
  if epad:
    ew = jnp.pad(ew, (0, epad), constant_values=(_NPAD - 1) * (1 << _SHIFT))

  sp = _segmax_call(ap, ew)
  s = lax.bitcast_convert_type(
      sp.reshape(_NPAD, _W), jnp.bfloat16).reshape(_NPAD, _C)

  out = _out_call(s, b, xp, Wg1, bg1.reshape(1, _C), Wg2, bg2.reshape(1, _C))
  return out[:n]


# speedup vs baseline: 7.1020x; 1.3390x over previous
import dataclasses
import functools

import jax
import jax.numpy as jnp
from jax import lax
from jax.experimental import pallas as pl
from jax.experimental.pallas import tpu as pltpu
from jax.experimental.pallas import tpu_sc as plsc

_NUM_CORES = 2
_NUM_SUBCORES = 16
_NW = _NUM_CORES * _NUM_SUBCORES
_LANES = 16

_C = 128
_W = _C // 2
_ROWS_PER_W = 320
_NPAD = _ROWS_PER_W * _NW
_CHUNK = 4096
_NQ = 4
_QE = _CHUNK // _NQ
_GB = 128
_QCAP = _QE + _GB + 16
_WLHALF = _NQ * _QCAP
_NEG = -3.0e38
_NEGBF2 = -8323200
_SHIFT = 14

_BLK = 1024


def _leaky(v):
  return jnp.where(v >= 0, v, 0.01 * v)


def _ab_body(x_ref, p8_ref, wfc_ref, wf38_ref, bf_ref, wh1_ref, bh1_ref,
             wh28_ref, bh28_ref, a_ref, b_ref):
  x = x_ref[...]
  p8 = p8_ref[...]
  h = _leaky(jnp.dot(x, wh1_ref[...], preferred_element_type=jnp.float32)
             + bh1_ref[...])
  d8 = jnp.tanh(jnp.dot(h, wh28_ref[...], preferred_element_type=jnp.float32)
                + bh28_ref[...])
  wf38 = wf38_ref[...]
  a_ref[...] = (jnp.dot(x, wfc_ref[...], preferred_element_type=jnp.float32)
                + jnp.dot(p8, wf38, preferred_element_type=jnp.float32)
                + bf_ref[...])
  b_ref[...] = jnp.dot(d8 - p8, wf38, preferred_element_type=jnp.float32)


def _ab_call(x, p8, wfc, wf38, bf2, wh1, bh12, wh28, bh28):
  n = x.shape[0]
  grid = (n // _BLK,)
  row = lambda i: (i, 0)
  fix = lambda i: (0, 0)
  return pl.pallas_call(
      _ab_body,
      grid=grid,
      in_specs=[
          pl.BlockSpec((_BLK, _C), row),
          pl.BlockSpec((_BLK, 8), row),
          pl.BlockSpec((_C, _C), fix),
          pl.BlockSpec((8, _C), fix),
          pl.BlockSpec((1, _C), fix),
          pl.BlockSpec((_C, _C), fix),
          pl.BlockSpec((1, _C), fix),
          pl.BlockSpec((_C, 8), fix),
          pl.BlockSpec((1, 8), fix),
      ],
      out_specs=[pl.BlockSpec((_BLK, _C), row), pl.BlockSpec((_BLK, _C), row)],
      out_shape=[jax.ShapeDtypeStruct((n, _C), jnp.float32)] * 2,
  )(x, p8, wfc, wf38, bf2, wh1, bh12, wh28, bh28)


def _segmax_kernel(a_hbm, ew_hbm, out_hbm,
                   ev, wl, st0, st1, st2, st3, stx,
                   rbuf0, rbuf1, rbuf2, rbuf3, rbufx, acc,
                   semc, semr0, semr1, semr2, semr3, semrx):
  wid = lax.axis_index("s") * _NUM_CORES + lax.axis_index("c")
  lo = wid * _ROWS_PER_W
  losh = lo * (1 << _SHIFT)
  rbufs = (rbuf0, rbuf1, rbuf2, rbuf3)
  semrs = (semr0, semr1, semr2, semr3)
  stages = (st0, st1, st2, st3)

  negp = jnp.full((_LANES,), _NEGBF2, jnp.int32)

  @pl.loop(0, (_ROWS_PER_W + 8) * _W, step=_LANES)
  def _(i):
    acc[pl.ds(i, _LANES)] = negp

  dums = jnp.full((_LANES,), _ROWS_PER_W * (1 << _SHIFT), jnp.int32)

  @pl.loop(0, 2 * _WLHALF, step=_LANES)
  def _(i):
    wl[pl.ds(i, _LANES)] = dums

  nchunks = ew_hbm.shape[0] // _CHUNK
  kiters = _QE // _LANES

  def fire_chunk(ci):
    off = lax.rem(ci, 2) * _CHUNK
    pltpu.async_copy(ew_hbm.at[pl.ds(ci * _CHUNK, _CHUNK)],
                     ev.at[pl.ds(off, _CHUNK)], semc)

  def wait_chunk():
    pltpu.make_async_copy(ew_hbm.at[pl.ds(0, _CHUNK)],
                          ev.at[pl.ds(0, _CHUNK)], semc).wait()

  def fire_batch(woff, stage, rbuf, semr):
    for k in range(_GB // _LANES):
      stage[pl.ds(k * _LANES, _LANES)] = (
          wl[pl.ds(woff + k * _LANES, _LANES)] & ((1 << _SHIFT) - 1))
    pltpu.async_copy(a_hbm.at[stage], rbuf, semr)

  def wait_batch(stage, rbuf, semr):
    pltpu.make_async_copy(a_hbm.at[stage], rbuf, semr).wait()

  def accum_batch(woff, rbuf):
    @pl.loop(0, _GB // _LANES)
    def _(g):
      dvec = lax.shift_right_logical(
          wl[pl.ds(woff + g * _LANES, _LANES)], _SHIFT)
      nsl = _W // _LANES
      for l in range(_LANES):
        base = dvec[l] * _W
        j = g * _LANES + l
        avs = [plsc.bitcast(acc[pl.ds(base + c * _LANES, _LANES)],
                            jnp.bfloat16) for c in range(nsl)]
        rvs = [plsc.bitcast(rbuf[j, pl.ds(c * _LANES, _LANES)],
                            jnp.bfloat16) for c in range(nsl)]
        mx = [jnp.maximum(avs[c], rvs[c]) for c in range(nsl)]
        for c in range(nsl):
          acc[pl.ds(base + c * _LANES, _LANES)] = plsc.bitcast(
              mx[c], jnp.int32)

  fire_chunk(0)

  thresh = jnp.uint32(_ROWS_PER_W * (1 << _SHIFT))

  def chunk_body(ci, carry):
    cnts, nfs = carry
    pw = lax.rem(ci, 2) * _WLHALF
    pv = lax.rem(ci + 1, 2) * _WLHALF
    wait_chunk()

    @pl.when(ci + 1 < nchunks)
    def _():
      fire_chunk(ci + 1)

    pbase = lax.rem(ci, 2) * _CHUNK

    def scan_iter(k, cvs):
      out = []
      for q in range(_NQ):
        w16 = ev[pl.ds(pbase + q * _QE + k * _LANES, _LANES)]
        t = w16 - losh
        mask = t.astype(jnp.uint32) < thresh
        mi = mask.astype(jnp.int32)
        csum = plsc.cumsum(mi)
        pos = csum + cvs[q]
        plsc.store_scatter(wl, [pos], t, mask=mask)
        out.append(cvs[q] + plsc.all_reduce_population_count(mask))
      return tuple(out)

    bases = [pw + q * _QCAP for q in range(_NQ)]
    cvs0 = tuple(
        jnp.full((_LANES,), 1, jnp.int32) * (bases[q] + cnts[q] - 1)
        for q in range(_NQ))
    cvs = plsc.parallel_loop(0, kiters, 1, unroll=2, carry=cvs0)(scan_iter)
    cnts = tuple(cvs[q][0] - (bases[q] - 1) for q in range(_NQ))

    new_cnts, new_nfs = [], []
    for q in range(_NQ):
      pwq = pw + q * _QCAP
      pvq = pv + q * _QCAP
      nb = cnts[q] // _GB

      @pl.when(nfs[q] > 0)
      def _():
        wait_batch(stages[q], rbufs[q], semrs[q])
        accum_batch(pvq, rbufs[q])

      @pl.when(nb > 0)
      def _():
        fire_batch(pwq, stages[q], rbufs[q], semrs[q])

      def extra(b, cc):
        fire_batch(pwq + b * _GB, stx, rbufx, semrx)
        wait_batch(stx, rbufx, semrx)
        accum_batch(pwq + b * _GB, rbufx)
        return cc

      lax.fori_loop(1, nb, extra, jnp.int32(0))

      for k in range(_GB // _LANES):
        wl[pl.ds(pvq + k * _LANES, _LANES)] = wl[pl.ds(
            pwq + nb * _GB + k * _LANES, _LANES)]
      new_cnts.append(cnts[q] - nb * _GB)
      new_nfs.append(jnp.minimum(nb, 1))

    return tuple(new_cnts), tuple(new_nfs)

  zero4 = (jnp.int32(0),) * _NQ
  cnts, nfs = lax.fori_loop(0, nchunks, chunk_body, (zero4, zero4))

  p_end = lax.rem(nchunks, 2) * _WLHALF
  p_last = lax.rem(nchunks + 1, 2) * _WLHALF
  for q in range(_NQ):
    @pl.when(nfs[q] > 0)
    def _():
      wait_batch(stages[q], rbufs[q], semrs[q])
      accum_batch(p_last + q * _QCAP, rbufs[q])
  for q in range(_NQ):
    fire_batch(p_end + q * _QCAP, stx, rbufx, semrx)
    wait_batch(stx, rbufx, semrx)
    accum_batch(p_end + q * _QCAP, rbufx)

  pltpu.sync_copy(acc.at[pl.ds(0, _ROWS_PER_W * _W)],
                  out_hbm.at[pl.ds(lo * _W, _ROWS_PER_W * _W)])


def _segmax_call(ap, ew):
  mesh = plsc.VectorSubcoreMesh(core_axis_name="c", subcore_axis_name="s")
  cp = pltpu.CompilerParams()
  if "needs_layout_passes" in pltpu.CompilerParams.__dataclass_fields__:
    cp = dataclasses.replace(cp, needs_layout_passes=False)
  if "use_tc_tiling_on_sc" in pltpu.CompilerParams.__dataclass_fields__:
    cp = dataclasses.replace(cp, use_tc_tiling_on_sc=False)
  kern = pl.kernel(
      _segmax_kernel,
      mesh=mesh,
      compiler_params=cp,
      out_type=jax.ShapeDtypeStruct((_NPAD * _W,), jnp.int32),
      scratch_types=[
          pltpu.VMEM((2 * _CHUNK,), jnp.int32),
          pltpu.VMEM((2 * _WLHALF,), jnp.int32),
          pltpu.VMEM((_GB,), jnp.int32),
          pltpu.VMEM((_GB,), jnp.int32),
          pltpu.VMEM((_GB,), jnp.int32),
          pltpu.VMEM((_GB,), jnp.int32),
          pltpu.VMEM((_GB,), jnp.int32),
          pltpu.VMEM((_GB, _W), jnp.int32),
          pltpu.VMEM((_GB, _W), jnp.int32),
          pltpu.VMEM((_GB, _W), jnp.int32),
          pltpu.VMEM((_GB, _W), jnp.int32),
          pltpu.VMEM((_GB, _W), jnp.int32),
          pltpu.VMEM(((_ROWS_PER_W + 8) * _W,), jnp.int32),
          pltpu.SemaphoreType.DMA,
          pltpu.SemaphoreType.DMA,
          pltpu.SemaphoreType.DMA,
          pltpu.SemaphoreType.DMA,
          pltpu.SemaphoreType.DMA,
          pltpu.SemaphoreType.DMA,
      ],
  )
  return kern(ap, ew)


def _out_body(s_ref, b_ref, x_ref, wg1_ref, bg1_ref, wg2_ref, bg2_ref, o_ref):
  s = s_ref[...].astype(jnp.float32)
  agg = _leaky(s + b_ref[...])
  agg = jnp.where(s < -1.0e38, 0.0, agg)
  g = _leaky(jnp.dot(agg, wg1_ref[...], preferred_element_type=jnp.float32)
             + bg1_ref[...])
  o_ref[...] = (x_ref[...]
                + jnp.dot(g, wg2_ref[...], preferred_element_type=jnp.float32)
                + bg2_ref[...])


def _out_call(s, b, x, wg1, bg12, wg2, bg22):
  n = x.shape[0]
  row = lambda i: (i, 0)
  fix = lambda i: (0, 0)
  return pl.pallas_call(
      _out_body,
      grid=(n // _BLK,),
      in_specs=[
          pl.BlockSpec((_BLK, _C), row),
          pl.BlockSpec((_BLK, _C), row),
          pl.BlockSpec((_BLK, _C), row),
          pl.BlockSpec((_C, _C), fix),
          pl.BlockSpec((1, _C), fix),
          pl.BlockSpec((_C, _C), fix),
          pl.BlockSpec((1, _C), fix),
      ],
      out_specs=pl.BlockSpec((_BLK, _C), row),
      out_shape=jax.ShapeDtypeStruct((n, _C), jnp.float32),
  )(s, b, x, wg1, bg12, wg2, bg22)


def kernel(x, pos, edge_index, Wh1, bh1, Wh2, bh2, Wf, bf, Wg1, bg1, Wg2, bg2):
  n = x.shape[0]
  e = edge_index.shape[1]

  xp = jnp.pad(x, ((0, _NPAD - n), (0, 0)))
  p8 = jnp.pad(pos, ((0, _NPAD - n), (0, 5)))
  wf38 = jnp.pad(Wf[:3], ((0, 5), (0, 0)))
  wfc = Wf[3:]
  wh28 = jnp.pad(Wh2, ((0, 0), (0, 5)))
  bh28 = jnp.pad(bh2, (0, 5)).reshape(1, 8)

  a, b = _ab_call(xp, p8, wfc, wf38, bf.reshape(1, _C), Wh1,
                  bh1.reshape(1, _C), wh28, bh28)

  ap = lax.bitcast_convert_type(
      a.astype(jnp.bfloat16).reshape(_NPAD, _W, 2), jnp.int32)

  src = edge_index[0].astype(jnp.int32)
  dst = edge_index[1].astype(jnp.int32)
  ew = dst * (1 << _SHIFT) + src
  epad = (-e) % _CHUNK
  if epad:
    ew = jnp.pad(ew, (0, epad), constant_values=(_NPAD - 1) * (1 << _SHIFT))

  sp = _segmax_call(ap, ew)
  s = lax.bitcast_convert_type(
      sp.reshape(_NPAD, _W), jnp.bfloat16).reshape(_NPAD, _C)

  out = _out_call(s, b, xp, Wg1, bg1.reshape(1, _C), Wg2, bg2.reshape(1, _C))
  return out[:n]
